# baseline (speedup 1.0000x reference)
import functools

import jax
import jax.numpy as jnp
from jax import lax
from jax.experimental import pallas as pl
from jax.experimental.pallas import tpu as pltpu
from jax.experimental.pallas import tpu_sc as plsc

NUM_INTENTS = 100000
EMBED_DIM = 64
BATCH = 16384
SEQ_LEN = 200

LANE = 128
NBT = BATCH // LANE
NW = 32
BT_PER_W = NBT // NW
UNITS = BT_PER_W * SEQ_LEN
NSLOT = 8
IDX_BYTES = LANE * 4
T_BYTES = EMBED_DIM * LANE * 4


def _gather_body(table_hbm, idxT_hbm, out_hbm, idx_v, rows_v, t_v, *sems):
    isems = sems[0:NSLOT]
    gsems = sems[NSLOT:2 * NSLOT]
    ssems = sems[2 * NSLOT:2 * NSLOT + 2]
    wid = lax.axis_index("s") * 2 + lax.axis_index("c")
    bt0 = wid * BT_PER_W

    iota = lax.iota(jnp.int32, 16)
    dconsts = []
    for d0 in range(0, EMBED_DIM, 16):
        dv = iota + d0
        dconsts.append((dv >> 3) * 128 + (dv & 7) * 16)

    def unit_su(u):
        return u % SEQ_LEN, bt0 + u // SEQ_LEN

    def fire_idx(u, p):
        s, bt = unit_su(u)
        pltpu.async_copy(idxT_hbm.at[s, pl.ds(bt * LANE, LANE)],
                         idx_v.at[p], isems[p])

    def wait_idx(p):
        pltpu.make_async_copy(idxT_hbm.at[0, pl.ds(0, LANE)],
                              idx_v.at[p], isems[p]).wait()

    def fire_gather(p):
        pltpu.async_copy(table_hbm.at[idx_v.at[p]], rows_v.at[p], gsems[p])

    def wait_gather(p):
        pltpu.make_async_copy(table_hbm.at[pl.ds(0, LANE)],
                              rows_v.at[p], gsems[p]).wait()

    def fire_store(u, q):
        s, bt = unit_su(u)
        for dt in range(8):
            pltpu.async_copy(t_v.at[q].at[pl.ds(dt * 128, 128)],
                             out_hbm.at[s * 8 + dt, bt], ssems[q])

    def wait_store(q):
        for dt in range(8):
            pltpu.make_async_copy(t_v.at[q].at[pl.ds(0, 128)],
                                  out_hbm.at[0, 0], ssems[q]).wait()

    def transpose(p, q):
        def tb(bl2, carry):
            for j in range(2):
                bl = 2 * bl2 + j
                vhi = jnp.full((16,), bl >> 3, jnp.int32)
                vlo = jnp.full((16,), bl & 7, jnp.int32)
                for k in range(EMBED_DIM // 16):
                    v = rows_v.at[p][bl, pl.ds(k * 16, 16)]
                    plsc.store_scatter(t_v.at[q], [dconsts[k] + vhi, vlo], v)
            return carry
        lax.fori_loop(0, LANE // 2, tb, 0)

    def step(u, p, do_wait_store):
        q = p % 2
        wait_gather(p)
        fire_idx(jnp.minimum(u + NSLOT, UNITS - 1), p)
        p4 = (p + NSLOT // 2) % NSLOT
        wait_idx(p4)
        fire_gather(p4)
        if do_wait_store:
            wait_store(q)
        transpose(p, q)
        fire_store(u, q)

    for p in range(NSLOT):
        fire_idx(p, p)
    for p in range(NSLOT // 2):
        wait_idx(p)
        fire_gather(p)
    for p in range(NSLOT):
        step(p, p, do_wait_store=p >= 2)

    def loop_body(tt, carry):
        for p in range(NSLOT):
            step(NSLOT * tt + p, p, do_wait_store=True)
        return carry

    lax.fori_loop(1, UNITS // NSLOT, loop_body, 0)

    for p in range(NSLOT // 2, NSLOT):
        wait_idx(p)
    for p in range(NSLOT // 2):
        wait_gather(p)
    wait_store(0)
    wait_store(1)


@jax.jit
def _gather(table, idxT):
    mesh = plsc.VectorSubcoreMesh(core_axis_name="c", subcore_axis_name="s")
    return pl.kernel(
        _gather_body,
        mesh=mesh,
        out_type=jax.ShapeDtypeStruct((SEQ_LEN * 8, NBT, LANE, 8),
                                      jnp.float32),
        scratch_types=[
            pltpu.VMEM((NSLOT, LANE), jnp.int32),
            pltpu.VMEM((NSLOT, LANE, EMBED_DIM), jnp.float32),
            pltpu.VMEM((2, 8 * LANE, 8), jnp.float32),
        ] + [pltpu.SemaphoreType.DMA] * (2 * NSLOT + 2),
        compiler_params=pltpu.CompilerParams(use_tc_tiling_on_sc=False,
                                             needs_layout_passes=False),
    )(table, idxT)


def kernel(intent_ids, table):
    idxT = intent_ids.T
    out4 = _gather(table, idxT)
    x5 = out4.reshape(SEQ_LEN, 8, NBT, 8, LANE)
    return x5.transpose(2, 4, 0, 1, 3).reshape(BATCH, SEQ_LEN, EMBED_DIM)


# speedup vs baseline: 9.7864x; 9.7864x over previous
import functools

import jax
import jax.numpy as jnp
from jax import lax
from jax.experimental import pallas as pl
from jax.experimental.pallas import tpu as pltpu
from jax.experimental.pallas import tpu_sc as plsc

NUM_INTENTS = 100000
EMBED_DIM = 64
BATCH = 16384
SEQ_LEN = 200

LANE = 128
NBT = BATCH // LANE
NW = 32
BT_PER_W = NBT // NW
UNITS = BT_PER_W * SEQ_LEN
NSLOT = 8
IDX_BYTES = LANE * 4
T_BYTES = EMBED_DIM * LANE * 4


def _gather_body(table_hbm, idxT_hbm, out_hbm, idx_v, rows_v, t_v, *sems):
    isems = sems[0:NSLOT]
    gsems = sems[NSLOT:2 * NSLOT]
    ssems = sems[2 * NSLOT:2 * NSLOT + 2]
    wid = lax.axis_index("s") * 2 + lax.axis_index("c")
    bt0 = wid * BT_PER_W

    iota = lax.iota(jnp.int32, 16)
    perms = [(iota + j) & 15 for j in range(16)]

    def unit_su(u):
        return u % SEQ_LEN, bt0 + u // SEQ_LEN

    def fire_idx(u, p):
        s, bt = unit_su(u)
        pltpu.async_copy(idxT_hbm.at[s, pl.ds(bt * LANE, LANE)],
                         idx_v.at[p], isems[p])

    def wait_idx(p):
        pltpu.make_async_copy(idxT_hbm.at[0, pl.ds(0, LANE)],
                              idx_v.at[p], isems[p]).wait()

    def fire_gather(p):
        pltpu.async_copy(table_hbm.at[idx_v.at[p]], rows_v.at[p], gsems[p])

    def wait_gather(p):
        pltpu.make_async_copy(table_hbm.at[pl.ds(0, LANE)],
                              rows_v.at[p], gsems[p]).wait()

    def fire_store(u, q):
        s, bt = unit_su(u)
        for dt in range(8):
            pltpu.async_copy(t_v.at[q].at[pl.ds(dt * 8, 8)],
                             out_hbm.at[s * 8 + dt, bt], ssems[q])

    def wait_store(q):
        for dt in range(8):
            pltpu.make_async_copy(t_v.at[q].at[pl.ds(0, 8)],
                                  out_hbm.at[0, 0], ssems[q]).wait()

    def transpose(p, q):
        def tb(i, carry):
            blc = iota + (i >> 2) * 16
            d0 = (i & 3) * 16
            for j in range(16):
                dvj = perms[j] + d0
                g = plsc.load_gather(rows_v.at[p], [blc, dvj])
                plsc.store_scatter(t_v.at[q], [dvj, blc], g)
            return carry
        lax.fori_loop(0, (LANE // 16) * (EMBED_DIM // 16), tb, 0)

    for p in range(NSLOT):
        fire_idx(p, p)
    for p in range(NSLOT // 2):
        wait_idx(p)
        fire_gather(p)

    def loop_body(tt, carry):
        for p in range(NSLOT):
            u = NSLOT * tt + p
            q = p % 2
            wait_gather(p)
            fire_idx(jnp.minimum(u + NSLOT, UNITS - 1), p)
            p4 = (p + NSLOT // 2) % NSLOT
            wait_idx(p4)
            fire_gather(p4)
            if p >= 2:
                wait_store(q)
            else:
                @pl.when(tt > 0)
                def _():
                    wait_store(q)
            transpose(p, q)
            fire_store(u, q)
        return carry

    lax.fori_loop(0, UNITS // NSLOT, loop_body, 0)

    for p in range(NSLOT // 2, NSLOT):
        wait_idx(p)
    for p in range(NSLOT // 2):
        wait_gather(p)
    wait_store(0)
    wait_store(1)


@jax.jit
def _gather(table, idxT):
    mesh = plsc.VectorSubcoreMesh(core_axis_name="c", subcore_axis_name="s")
    return pl.kernel(
        _gather_body,
        mesh=mesh,
        out_type=jax.ShapeDtypeStruct((SEQ_LEN * 8, NBT, 8, LANE),
                                      jnp.float32),
        scratch_types=[
            pltpu.VMEM((NSLOT, LANE), jnp.int32),
            pltpu.VMEM((NSLOT, LANE, EMBED_DIM), jnp.float32),
            pltpu.VMEM((2, EMBED_DIM, LANE), jnp.float32),
        ] + [pltpu.SemaphoreType.DMA] * (2 * NSLOT + 2),
        compiler_params=pltpu.CompilerParams(use_tc_tiling_on_sc=False,
                                             needs_layout_passes=False),
    )(table, idxT)


def kernel(intent_ids, table):
    idxT = intent_ids.T
    out4 = _gather(table, idxT)
    x5 = out4.reshape(SEQ_LEN, 8, NBT, 8, LANE)
    return x5.transpose(2, 4, 0, 1, 3).reshape(BATCH, SEQ_LEN, EMBED_DIM)
